# baseline (speedup 1.0000x reference)
import functools

import jax
import jax.numpy as jnp
from jax import lax
from jax.experimental import pallas as pl
from jax.experimental.pallas import tpu as pltpu
from jax.experimental.pallas import tpu_sc as plsc

N = 10000
E = 320000
D = 128
DH = D // 2
NC, NS = 2, 16
CHUNK = 128
NCH = 160
EPAD = NS * NCH * CHUNK
ACC_ROWS = 10240
DUMMY = 10008
BR = 2000
K = 4
IR = 8

_MESH = plsc.VectorSubcoreMesh(
    core_axis_name="c", subcore_axis_name="s", num_cores=NC, num_subcores=NS)

_Z16 = functools.partial(jnp.zeros, (16,), jnp.float32)


def _zero_vmem(ref, rows, cols):
    vper = cols // 16
    def body(i, _):
        ref[i // vper, pl.ds((i % vper) * 16, 16)] = _Z16()
        return 0
    lax.fori_loop(0, rows * vper, body, 0)


def _zero_vmem_1d(ref, nwords):
    def body(i, _):
        ref[pl.ds(i * 16, 16)] = _Z16()
        return 0
    lax.fori_loop(0, nwords // 16, body, 0)


@functools.partial(
    pl.kernel,
    out_type=jax.ShapeDtypeStruct((NC, ACC_ROWS, DH), jnp.float32),
    mesh=_MESH,
    compiler_params=pltpu.CompilerParams(use_tc_tiling_on_sc=False),
    scratch_types=[
        pltpu.VMEM((IR, 2, CHUNK), jnp.int32),
        pltpu.VMEM((K, CHUNK, DH), jnp.float32),
        pltpu.VMEM_SHARED((ACC_ROWS, DH), jnp.float32),
        pltpu.VMEM_SHARED((ACC_ROWS, DH), jnp.float32),
        pltpu.SemaphoreType.DMA((IR,)),
        pltpu.SemaphoreType.DMA((K,)),
        pltpu.SemaphoreType.DMA((K,)),
    ],
)
def _sc_agg(h, idx_i, out, idxr, rows, h_sp, acc, semi, semg, sems):
    c = lax.axis_index("c")
    s = lax.axis_index("s")
    seg = ACC_ROWS // NS

    for j in range(IR):
        pltpu.async_copy(idx_i.at[s, j], idxr.at[j], semi.at[j])

    _zero_vmem(rows.at[0], CHUNK, DH)
    def zacc(k, _):
        pltpu.sync_copy(rows.at[0], acc.at[pl.ds((s * 5 + k) * CHUNK, CHUNK)])
        return 0
    lax.fori_loop(0, ACC_ROWS // (NS * CHUNK), zacc, 0)

    hh = h.at[c]
    @pl.when(s < NS - 1)
    def _():
        pltpu.sync_copy(hh.at[pl.ds(s * seg, seg)],
                        h_sp.at[pl.ds(s * seg, seg)])
    @pl.when(s == NS - 1)
    def _():
        pltpu.sync_copy(hh.at[pl.ds(N - seg, seg)],
                        h_sp.at[pl.ds(N - seg, seg)])
    plsc.subcore_barrier()

    def gather(cc, slot, j):
        @pl.when(cc % 16 < 7)
        def _():
            pltpu.async_copy(hh.at[idxr.at[slot, 0]], rows.at[j], semg.at[j])
        @pl.when(cc % 16 >= 7)
        def _():
            pltpu.async_copy(h_sp.at[idxr.at[slot, 0]], rows.at[j],
                             semg.at[j])

    for j in range(K):
        pltpu.make_async_copy(idx_i.at[s, j], idxr.at[j], semi.at[j]).wait()
        gather(j, j, j)
    nsteps = NCH // K

    def step(g, _):
        c0 = g * K
        for j in range(K):
            cc = c0 + j
            slot = cc % IR
            pltpu.make_async_copy(
                h_sp.at[idxr.at[slot, 0]], rows.at[j], semg.at[j]).wait()
            pltpu.async_copy(
                rows.at[j], acc.at[idxr.at[slot, 1]], sems.at[j], add=True)
        for j in range(K):
            cc = c0 + j
            slot = cc % IR
            pltpu.make_async_copy(
                rows.at[j], acc.at[idxr.at[slot, 1]], sems.at[j]).wait()

            @pl.when(cc + IR < NCH)
            def _():
                pltpu.async_copy(idx_i.at[s, cc + IR], idxr.at[slot],
                                 semi.at[slot])

            @pl.when(cc + K < NCH)
            def _():
                slot2 = (cc + K) % IR
                pltpu.make_async_copy(idx_i.at[s, cc + K], idxr.at[slot2],
                                      semi.at[slot2]).wait()
                gather(cc + K, slot2, j)
        return 0

    lax.fori_loop(0, nsteps, step, 0)
    plsc.subcore_barrier()
    pltpu.sync_copy(acc.at[pl.ds(s * seg, seg)],
                    out.at[c, pl.ds(s * seg, seg)])


@functools.partial(
    pl.kernel,
    out_type=jax.ShapeDtypeStruct((NC, ACC_ROWS), jnp.float32),
    mesh=_MESH,
    scratch_types=[
        pltpu.VMEM((NCH, 2, CHUNK), jnp.int32),
        pltpu.VMEM((CHUNK,), jnp.float32),
        pltpu.VMEM((ACC_ROWS // NS,), jnp.float32),
        pltpu.VMEM_SHARED((ACC_ROWS,), jnp.float32),
    ],
)
def _sc_deg(idx_i, out, idx_v, ones_v, zvec, acc):
    c = lax.axis_index("c")
    s = lax.axis_index("s")
    seg = ACC_ROWS // NS

    for j in range(CHUNK // 16):
        ones_v[pl.ds(j * 16, 16)] = jnp.ones((16,), jnp.float32)
    _zero_vmem_1d(zvec, seg)
    pltpu.sync_copy(zvec, acc.at[pl.ds(s * seg, seg)])

    pltpu.sync_copy(idx_i.at[s], idx_v)
    plsc.subcore_barrier()

    def step(g, _):
        pltpu.sync_copy(ones_v, acc.at[idx_v.at[g, 1]], add=True)
        return 0
    lax.fori_loop(c * (NCH // 2), (c + 1) * (NCH // 2), step, 0)

    plsc.subcore_barrier()
    pltpu.sync_copy(acc.at[pl.ds(s * seg, seg)], out.at[c, pl.ds(s * seg, seg)])


@functools.partial(
    pl.kernel,
    out_type=jax.ShapeDtypeStruct((NC, ACC_ROWS), jnp.float32),
    mesh=_MESH,
    compiler_params=pltpu.CompilerParams(use_tc_tiling_on_sc=False),
    scratch_types=[
        pltpu.VMEM((IR, 2, CHUNK), jnp.int32),
        pltpu.VMEM((K, CHUNK), jnp.float32),
        pltpu.VMEM((ACC_ROWS // NS,), jnp.float32),
        pltpu.VMEM_SHARED((ACC_ROWS,), jnp.float32),
        pltpu.VMEM_SHARED((ACC_ROWS,), jnp.float32),
        pltpu.SemaphoreType.DMA((IR,)),
        pltpu.SemaphoreType.DMA((K,)),
        pltpu.SemaphoreType.DMA((K,)),
    ],
)
def _sc_agg1(g3, idx_i, out, idxr, rows, zvec, g3_sp, acc, semi, semg, sems):
    c = lax.axis_index("c")
    s = lax.axis_index("s")
    seg = ACC_ROWS // NS
    half = NCH // 2
    base = c * half

    for j in range(IR):
        pltpu.async_copy(idx_i.at[s, base + j], idxr.at[j], semi.at[j])

    _zero_vmem_1d(zvec, seg)
    pltpu.sync_copy(zvec, acc.at[pl.ds(s * seg, seg)])
    @pl.when(s == 0)
    def _():
        pltpu.sync_copy(g3, g3_sp.at[pl.ds(0, N)])
    plsc.subcore_barrier()

    for j in range(K):
        pltpu.make_async_copy(
            idx_i.at[s, base + j], idxr.at[j], semi.at[j]).wait()
        pltpu.async_copy(g3_sp.at[idxr.at[j, 0]], rows.at[j], semg.at[j])
    nsteps = half // K

    def step(g, _):
        c0 = g * K
        for j in range(K):
            cr = c0 + j
            slot = cr % IR
            pltpu.make_async_copy(
                g3_sp.at[idxr.at[slot, 0]], rows.at[j], semg.at[j]).wait()
            pltpu.async_copy(
                rows.at[j], acc.at[idxr.at[slot, 1]], sems.at[j], add=True)
        for j in range(K):
            cr = c0 + j
            slot = cr % IR
            pltpu.make_async_copy(
                rows.at[j], acc.at[idxr.at[slot, 1]], sems.at[j]).wait()

            @pl.when(cr + IR < half)
            def _():
                pltpu.async_copy(idx_i.at[s, base + cr + IR], idxr.at[slot],
                                 semi.at[slot])

            @pl.when(cr + K < half)
            def _():
                slot2 = (cr + K) % IR
                pltpu.make_async_copy(idx_i.at[s, base + cr + K],
                                      idxr.at[slot2], semi.at[slot2]).wait()
                pltpu.async_copy(g3_sp.at[idxr.at[slot2, 0]], rows.at[j],
                                 semg.at[j])
        return 0

    lax.fori_loop(0, nsteps, step, 0)
    plsc.subcore_barrier()
    pltpu.sync_copy(acc.at[pl.ds(s * seg, seg)], out.at[c, pl.ds(s * seg, seg)])


def _k1_body(x_ref, w_ref, dp_ref, g_ref, dis_ref):
    dis = lax.rsqrt(dp_ref[0] + dp_ref[1] + 1.0)
    g = jnp.dot(x_ref[...], w_ref[...], preferred_element_type=jnp.float32)
    g_ref[0] = g[:, :DH] * dis
    g_ref[1] = g[:, DH:] * dis
    dis_ref[...] = dis


def _k1(x, w, dp):
    return pl.pallas_call(
        _k1_body,
        grid=(N // BR,),
        in_specs=[
            pl.BlockSpec((BR, D), lambda i: (i, 0)),
            pl.BlockSpec((D, D), lambda i: (0, 0)),
            pl.BlockSpec((NC, BR, 1), lambda i: (0, i, 0)),
        ],
        out_specs=[pl.BlockSpec((NC, BR, DH), lambda i: (0, i, 0)),
                   pl.BlockSpec((BR, 1), lambda i: (i, 0))],
        out_shape=[jax.ShapeDtypeStruct((NC, N, DH), jnp.float32),
                   jax.ShapeDtypeStruct((N, 1), jnp.float32)],
    )(x, w, dp)


def _relu_h(p_ref, g_ref, dis_ref, b_ref):
    dis = dis_ref[...]
    z0 = dis * (p_ref[0] + g_ref[0]) + b_ref[0, :DH]
    z1 = dis * (p_ref[1] + g_ref[1]) + b_ref[0, DH:]
    return jnp.concatenate(
        [jnp.maximum(z0, 0.0), jnp.maximum(z1, 0.0)], axis=1)


def _k2_body(p_ref, g_ref, dis_ref, b_ref, w_ref, o_ref):
    h = _relu_h(p_ref, g_ref, dis_ref, b_ref)
    g = jnp.dot(h, w_ref[...], preferred_element_type=jnp.float32)
    dis = dis_ref[...]
    o_ref[0] = g[:, :DH] * dis
    o_ref[1] = g[:, DH:] * dis


def _k3_body(p_ref, g_ref, dis_ref, b_ref, w_ref, o_ref):
    h = _relu_h(p_ref, g_ref, dis_ref, b_ref)
    g = jnp.dot(h, w_ref[...], preferred_element_type=jnp.float32)
    o_ref[...] = g * dis_ref[...]


def _k23(body, p, g, dis, b, w, out_shape, out_spec):
    return pl.pallas_call(
        body,
        grid=(N // BR,),
        in_specs=[
            pl.BlockSpec((NC, BR, DH), lambda i: (0, i, 0)),
            pl.BlockSpec((NC, BR, DH), lambda i: (0, i, 0)),
            pl.BlockSpec((BR, 1), lambda i: (i, 0)),
            pl.BlockSpec((1, D), lambda i: (0, 0)),
            pl.BlockSpec(w.shape, lambda i: (0, 0)),
        ],
        out_specs=out_spec,
        out_shape=out_shape,
    )(p, g, dis, b, w)


def _k4_body(r_ref, g_ref, dis_ref, b_ref, o_ref):
    o_ref[...] = dis_ref[...] * (r_ref[0] + r_ref[1] + g_ref[...]) + b_ref[...]


def _k4(r, g, dis, b):
    return pl.pallas_call(
        _k4_body,
        grid=(N // BR,),
        in_specs=[
            pl.BlockSpec((NC, BR, 1), lambda i: (0, i, 0)),
            pl.BlockSpec((BR, 1), lambda i: (i, 0)),
            pl.BlockSpec((BR, 1), lambda i: (i, 0)),
            pl.BlockSpec((1, 1), lambda i: (0, 0)),
        ],
        out_specs=pl.BlockSpec((BR, 1), lambda i: (i, 0)),
        out_shape=jax.ShapeDtypeStruct((N, 1), jnp.float32),
    )(r, g, dis, b)


def kernel(x, edge_index, y, W1, b1, W2, b2, W3, b3):
    src = edge_index[0].astype(jnp.int32)
    dst = edge_index[1].astype(jnp.int32)
    pad = EPAD - E
    src3 = jnp.concatenate([src, jnp.zeros((pad,), jnp.int32)]).reshape(
        NS, NCH, CHUNK)
    dst3 = jnp.concatenate([dst, jnp.full((pad,), DUMMY, jnp.int32)]).reshape(
        NS, NCH, CHUNK)
    idx3 = jnp.stack([src3, dst3], axis=2)

    degp = _sc_deg(idx3).reshape(NC, ACC_ROWS, 1)
    g1, dis = _k1(x, W1, degp)
    p1 = _sc_agg(g1, idx3)
    g2 = _k23(_k2_body, p1, g1, dis, b1.reshape(1, D), W2,
              jax.ShapeDtypeStruct((NC, N, DH), jnp.float32),
              pl.BlockSpec((NC, BR, DH), lambda i: (0, i, 0)))
    p2 = _sc_agg(g2, idx3)
    g3 = _k23(_k3_body, p2, g2, dis, b2.reshape(1, D), W3,
              jax.ShapeDtypeStruct((N, 1), jnp.float32),
              pl.BlockSpec((BR, 1), lambda i: (i, 0)))
    p3 = _sc_agg1(g3.reshape(N), idx3)
    return _k4(p3.reshape(NC, ACC_ROWS, 1), g3, dis, b3.reshape(1, 1))


# speedup vs baseline: 1.1635x; 1.1635x over previous
import functools

import jax
import jax.numpy as jnp
from jax import lax
from jax.experimental import pallas as pl
from jax.experimental.pallas import tpu as pltpu
from jax.experimental.pallas import tpu_sc as plsc

N = 10000
E = 320000
D = 128
DH = D // 2
NC, NS = 2, 16
CHUNK = 128
NCH = 160
EPAD = NS * NCH * CHUNK
ACC_ROWS = 10240
DUMMY = 10008
BR = 2000
K = 4
IR = 8

_MESH = plsc.VectorSubcoreMesh(
    core_axis_name="c", subcore_axis_name="s", num_cores=NC, num_subcores=NS)

_Z16 = functools.partial(jnp.zeros, (16,), jnp.float32)


def _zero_vmem(ref, rows, cols):
    vper = cols // 16
    def body(i, _):
        ref[i // vper, pl.ds((i % vper) * 16, 16)] = _Z16()
        return 0
    lax.fori_loop(0, rows * vper, body, 0)


def _zero_vmem_1d(ref, nwords):
    def body(i, _):
        ref[pl.ds(i * 16, 16)] = _Z16()
        return 0
    lax.fori_loop(0, nwords // 16, body, 0)


@functools.partial(
    pl.kernel,
    out_type=jax.ShapeDtypeStruct((NC, ACC_ROWS, DH), jnp.float32),
    mesh=_MESH,
    compiler_params=pltpu.CompilerParams(use_tc_tiling_on_sc=False),
    scratch_types=[
        pltpu.VMEM((IR, 2, CHUNK), jnp.int32),
        pltpu.VMEM((K, CHUNK, DH), jnp.float32),
        pltpu.VMEM_SHARED((ACC_ROWS, DH), jnp.float32),
        pltpu.VMEM_SHARED((ACC_ROWS, DH), jnp.float32),
        pltpu.SemaphoreType.DMA((IR,)),
        pltpu.SemaphoreType.DMA((K,)),
        pltpu.SemaphoreType.DMA((K,)),
    ],
)
def _sc_agg(h, idx_i, out, idxr, rows, h_sp, acc, semi, semg, sems):
    c = lax.axis_index("c")
    s = lax.axis_index("s")
    seg = ACC_ROWS // NS

    for j in range(IR):
        pltpu.async_copy(idx_i.at[s, j], idxr.at[j], semi.at[j])

    _zero_vmem(rows.at[0], CHUNK, DH)
    def zacc(k, _):
        pltpu.sync_copy(rows.at[0], acc.at[pl.ds((s * 5 + k) * CHUNK, CHUNK)])
        return 0
    lax.fori_loop(0, ACC_ROWS // (NS * CHUNK), zacc, 0)

    hh = h.at[c]
    @pl.when(s < NS - 1)
    def _():
        pltpu.sync_copy(hh.at[pl.ds(s * seg, seg)],
                        h_sp.at[pl.ds(s * seg, seg)])
    @pl.when(s == NS - 1)
    def _():
        pltpu.sync_copy(hh.at[pl.ds(N - seg, seg)],
                        h_sp.at[pl.ds(N - seg, seg)])
    plsc.subcore_barrier()

    for j in range(K):
        pltpu.make_async_copy(idx_i.at[s, j], idxr.at[j], semi.at[j]).wait()
        pltpu.async_copy(h_sp.at[idxr.at[j, 0]], rows.at[j], semg.at[j])
    nsteps = NCH // K

    def step(g, _):
        c0 = g * K
        for j in range(K):
            cc = c0 + j
            slot = cc % IR
            pltpu.make_async_copy(
                h_sp.at[idxr.at[slot, 0]], rows.at[j], semg.at[j]).wait()
            pltpu.async_copy(
                rows.at[j], acc.at[idxr.at[slot, 1]], sems.at[j], add=True)
        for j in range(K):
            cc = c0 + j
            slot = cc % IR
            pltpu.make_async_copy(
                rows.at[j], acc.at[idxr.at[slot, 1]], sems.at[j]).wait()

            @pl.when(cc + IR < NCH)
            def _():
                pltpu.async_copy(idx_i.at[s, cc + IR], idxr.at[slot],
                                 semi.at[slot])

            @pl.when(cc + K < NCH)
            def _():
                slot2 = (cc + K) % IR
                pltpu.make_async_copy(idx_i.at[s, cc + K], idxr.at[slot2],
                                      semi.at[slot2]).wait()
                pltpu.async_copy(h_sp.at[idxr.at[slot2, 0]], rows.at[j],
                                 semg.at[j])
        return 0

    lax.fori_loop(0, nsteps, step, 0)
    plsc.subcore_barrier()
    pltpu.sync_copy(acc.at[pl.ds(s * seg, seg)],
                    out.at[c, pl.ds(s * seg, seg)])


@functools.partial(
    pl.kernel,
    out_type=jax.ShapeDtypeStruct((NC, ACC_ROWS), jnp.float32),
    mesh=_MESH,
    scratch_types=[
        pltpu.VMEM((NCH, 2, CHUNK), jnp.int32),
        pltpu.VMEM((CHUNK,), jnp.float32),
        pltpu.VMEM((ACC_ROWS // NS,), jnp.float32),
        pltpu.VMEM_SHARED((ACC_ROWS,), jnp.float32),
    ],
)
def _sc_deg(idx_i, out, idx_v, ones_v, zvec, acc):
    c = lax.axis_index("c")
    s = lax.axis_index("s")
    seg = ACC_ROWS // NS

    for j in range(CHUNK // 16):
        ones_v[pl.ds(j * 16, 16)] = jnp.ones((16,), jnp.float32)
    _zero_vmem_1d(zvec, seg)
    pltpu.sync_copy(zvec, acc.at[pl.ds(s * seg, seg)])

    pltpu.sync_copy(idx_i.at[s], idx_v)
    plsc.subcore_barrier()

    def step(g, _):
        pltpu.sync_copy(ones_v, acc.at[idx_v.at[g, 1]], add=True)
        return 0
    lax.fori_loop(c * (NCH // 2), (c + 1) * (NCH // 2), step, 0)

    plsc.subcore_barrier()
    pltpu.sync_copy(acc.at[pl.ds(s * seg, seg)], out.at[c, pl.ds(s * seg, seg)])


_SLAB = ACC_ROWS // (NC * NS)


@functools.partial(
    pl.kernel,
    out_type=jax.ShapeDtypeStruct((ACC_ROWS,), jnp.float32),
    mesh=_MESH,
    compiler_params=pltpu.CompilerParams(use_tc_tiling_on_sc=False),
    scratch_types=[
        pltpu.VMEM((IR, 2, CHUNK), jnp.int32),
        pltpu.VMEM((K, CHUNK), jnp.float32),
        pltpu.VMEM((ACC_ROWS // NS,), jnp.float32),
        pltpu.VMEM((_SLAB,), jnp.float32),
        pltpu.VMEM((_SLAB,), jnp.float32),
        pltpu.VMEM((_SLAB,), jnp.float32),
        pltpu.VMEM((16,), jnp.float32),
        pltpu.VMEM_SHARED((ACC_ROWS,), jnp.float32),
        pltpu.VMEM_SHARED((ACC_ROWS,), jnp.float32),
        pltpu.SemaphoreType.DMA((IR,)),
        pltpu.SemaphoreType.DMA((K,)),
        pltpu.SemaphoreType.DMA((K,)),
    ],
)
def _sc_agg1(g3, dis, b3, idx_i, out, idxr, rows, zvec, disv, g3v, ov, b3v,
             g3_sp, acc, semi, semg, sems):
    c = lax.axis_index("c")
    s = lax.axis_index("s")
    seg = ACC_ROWS // NS

    for j in range(IR):
        pltpu.async_copy(idx_i.at[s, j], idxr.at[j], semi.at[j])

    _zero_vmem_1d(zvec, seg)
    pltpu.sync_copy(zvec, acc.at[pl.ds(s * seg, seg)])
    @pl.when(s == 0)
    def _():
        pltpu.sync_copy(g3, g3_sp)
    plsc.subcore_barrier()

    for j in range(K):
        pltpu.make_async_copy(idx_i.at[s, j], idxr.at[j], semi.at[j]).wait()
        pltpu.async_copy(g3_sp.at[idxr.at[j, 0]], rows.at[j], semg.at[j])
    nsteps = NCH // K

    def step(g, _):
        c0 = g * K
        for j in range(K):
            cr = c0 + j
            slot = cr % IR
            pltpu.make_async_copy(
                g3_sp.at[idxr.at[slot, 0]], rows.at[j], semg.at[j]).wait()
            pltpu.async_copy(
                rows.at[j], acc.at[idxr.at[slot, 1]], sems.at[j], add=True)
        for j in range(K):
            cr = c0 + j
            slot = cr % IR
            pltpu.make_async_copy(
                rows.at[j], acc.at[idxr.at[slot, 1]], sems.at[j]).wait()

            @pl.when(cr + IR < NCH)
            def _():
                pltpu.async_copy(idx_i.at[s, cr + IR], idxr.at[slot],
                                 semi.at[slot])

            @pl.when(cr + K < NCH)
            def _():
                slot2 = (cr + K) % IR
                pltpu.make_async_copy(idx_i.at[s, cr + K],
                                      idxr.at[slot2], semi.at[slot2]).wait()
                pltpu.async_copy(g3_sp.at[idxr.at[slot2, 0]], rows.at[j],
                                 semg.at[j])
        return 0

    lax.fori_loop(0, nsteps, step, 0)
    plsc.subcore_barrier()

    base = (c * NS + s) * _SLAB
    pltpu.sync_copy(acc.at[pl.ds(base, _SLAB)], ov)
    pltpu.sync_copy(dis.at[pl.ds(base, _SLAB)], disv)
    pltpu.sync_copy(g3.at[pl.ds(base, _SLAB)], g3v)
    pltpu.sync_copy(b3, b3v)
    bvec = b3v[...]
    for i in range(_SLAB // 16):
        sl = pl.ds(i * 16, 16)
        ov[sl] = disv[sl] * (ov[sl] + g3v[sl]) + bvec
    pltpu.sync_copy(ov, out.at[pl.ds(base, _SLAB)])


def _k1_body(x_ref, w_ref, dp_ref, g_ref, dis_ref):
    dis = lax.rsqrt(dp_ref[0] + dp_ref[1] + 1.0)
    g = jnp.dot(x_ref[...], w_ref[...], preferred_element_type=jnp.float32)
    g_ref[0] = g[:, :DH] * dis
    g_ref[1] = g[:, DH:] * dis
    dis_ref[...] = dis


def _k1(x, w, dp):
    return pl.pallas_call(
        _k1_body,
        grid=(N // BR,),
        in_specs=[
            pl.BlockSpec((BR, D), lambda i: (i, 0)),
            pl.BlockSpec((D, D), lambda i: (0, 0)),
            pl.BlockSpec((NC, BR, 1), lambda i: (0, i, 0)),
        ],
        out_specs=[pl.BlockSpec((NC, BR, DH), lambda i: (0, i, 0)),
                   pl.BlockSpec((BR, 1), lambda i: (i, 0))],
        out_shape=[jax.ShapeDtypeStruct((NC, N, DH), jnp.float32),
                   jax.ShapeDtypeStruct((N, 1), jnp.float32)],
    )(x, w, dp)


def _relu_h(p_ref, g_ref, dis_ref, b_ref):
    dis = dis_ref[...]
    z0 = dis * (p_ref[0] + g_ref[0]) + b_ref[0, :DH]
    z1 = dis * (p_ref[1] + g_ref[1]) + b_ref[0, DH:]
    return jnp.concatenate(
        [jnp.maximum(z0, 0.0), jnp.maximum(z1, 0.0)], axis=1)


def _k2_body(p_ref, g_ref, dis_ref, b_ref, w_ref, o_ref):
    h = _relu_h(p_ref, g_ref, dis_ref, b_ref)
    g = jnp.dot(h, w_ref[...], preferred_element_type=jnp.float32)
    dis = dis_ref[...]
    o_ref[0] = g[:, :DH] * dis
    o_ref[1] = g[:, DH:] * dis


def _k3_body(p_ref, g_ref, dis_ref, b_ref, w_ref, o_ref):
    h = _relu_h(p_ref, g_ref, dis_ref, b_ref)
    g = jnp.dot(h, w_ref[...], preferred_element_type=jnp.float32)
    o_ref[...] = g * dis_ref[...]


def _k23(body, p, g, dis, b, w, out_shape, out_spec):
    return pl.pallas_call(
        body,
        grid=(N // BR,),
        in_specs=[
            pl.BlockSpec((NC, BR, DH), lambda i: (0, i, 0)),
            pl.BlockSpec((NC, BR, DH), lambda i: (0, i, 0)),
            pl.BlockSpec((BR, 1), lambda i: (i, 0)),
            pl.BlockSpec((1, D), lambda i: (0, 0)),
            pl.BlockSpec(w.shape, lambda i: (0, 0)),
        ],
        out_specs=out_spec,
        out_shape=out_shape,
    )(p, g, dis, b, w)


def kernel(x, edge_index, y, W1, b1, W2, b2, W3, b3):
    src = edge_index[0].astype(jnp.int32)
    dst = edge_index[1].astype(jnp.int32)
    pad = EPAD - E
    src3 = jnp.concatenate([src, jnp.zeros((pad,), jnp.int32)]).reshape(
        NS, NCH, CHUNK)
    dst3 = jnp.concatenate([dst, jnp.full((pad,), DUMMY, jnp.int32)]).reshape(
        NS, NCH, CHUNK)
    idx3 = jnp.stack([src3, dst3], axis=2)

    degp = _sc_deg(idx3).reshape(NC, ACC_ROWS, 1)
    g1, dis = _k1(x, W1, degp)
    p1 = _sc_agg(g1, idx3)
    g2 = _k23(_k2_body, p1, g1, dis, b1.reshape(1, D), W2,
              jax.ShapeDtypeStruct((NC, N, DH), jnp.float32),
              pl.BlockSpec((NC, BR, DH), lambda i: (0, i, 0)))
    p2 = _sc_agg(g2, idx3)
    g3 = _k23(_k3_body, p2, g2, dis, b2.reshape(1, D), W3,
              jax.ShapeDtypeStruct((N, 1), jnp.float32),
              pl.BlockSpec((BR, 1), lambda i: (i, 0)))
    zpad = jnp.zeros((ACC_ROWS - N,), jnp.float32)
    g3p = jnp.concatenate([g3.reshape(N), zpad])
    disp = jnp.concatenate([dis.reshape(N), zpad])
    b3v = jnp.full((16,), b3[0], jnp.float32)
    outp = _sc_agg1(g3p, disp, b3v, idx3)
    return outp[:N].reshape(N, 1)
